# baseline (speedup 1.0000x reference)
import functools

import jax
import jax.numpy as jnp
from jax.experimental import pallas as pl
from jax.experimental.pallas import tpu as pltpu

_E = 8
_C = 10
_DIN = 768
_DF = 768
_DOUT = 64
_TB = 2048


def _fused_body(state_ref, c_ref, wg_ref, w1_ref, b1_ref, w2_ref, b2_ref,
                hw_ref, hb_ref, out_ref, acc_ref):
    e = pl.program_id(1)

    x = state_ref[...]
    h = jnp.dot(x, w1_ref[0], preferred_element_type=jnp.float32)
    h = jnp.maximum(h + b1_ref[0], 0.0).astype(jnp.bfloat16)
    f = jnp.dot(h, w2_ref[0], preferred_element_type=jnp.float32)
    f = jnp.maximum(f + b2_ref[0], 0.0)

    c_col = c_ref[0]
    gate = wg_ref[0, 0]

    contrib = gate * f

    @pl.when(e == 0)
    def _init():
        acc_ref[...] = contrib

    @pl.when(e > 0)
    def _accum():
        acc_ref[...] += contrib

    @pl.when(e == _E - 1)
    def _heads():
        mixed = jnp.maximum(acc_ref[...], 0.0).astype(jnp.bfloat16)
        all_heads = jnp.dot(mixed, hw_ref[...],
                            preferred_element_type=jnp.float32)
        all_heads = all_heads + hb_ref[...]
        out = jnp.zeros((_TB, _DOUT), jnp.float32)
        for ci in range(_C):
            sl = all_heads[:, ci * _DOUT:(ci + 1) * _DOUT]
            out = out + jnp.where(c_col == ci, sl, 0.0)
        out_ref[...] = out


@functools.partial(jax.jit, static_argnames=())
def kernel(state, c, W_task, W1, b1, W2, b2, head_W, head_b):
    B = state.shape[0]
    nb = B // _TB
    c3 = c.astype(jnp.int32).reshape(nb, _TB, 1)
    wg = W_task.T[c.astype(jnp.int32)]
    wg = wg.T.reshape(_E, nb, _TB, 1).transpose(1, 0, 2, 3)
    hw_cat = jnp.transpose(head_W, (1, 0, 2)).reshape(_DF, _C * _DOUT)
    hb_cat = head_b.reshape(1, _C * _DOUT)

    out = pl.pallas_call(
        _fused_body,
        grid=(nb, _E),
        in_specs=[
            pl.BlockSpec((_TB, _DIN), lambda ib, e: (ib, 0)),
            pl.BlockSpec((1, _TB, 1), lambda ib, e: (ib, 0, 0)),
            pl.BlockSpec((1, 1, _TB, 1), lambda ib, e: (ib, e, 0, 0)),
            pl.BlockSpec((1, _DIN, _DF), lambda ib, e: (e, 0, 0)),
            pl.BlockSpec((1, 1, _DF), lambda ib, e: (e, 0, 0)),
            pl.BlockSpec((1, _DF, _DF), lambda ib, e: (e, 0, 0)),
            pl.BlockSpec((1, 1, _DF), lambda ib, e: (e, 0, 0)),
            pl.BlockSpec((_DF, _C * _DOUT), lambda ib, e: (0, 0)),
            pl.BlockSpec((1, _C * _DOUT), lambda ib, e: (0, 0)),
        ],
        out_specs=pl.BlockSpec((_TB, _DOUT), lambda ib, e: (ib, 0)),
        out_shape=jax.ShapeDtypeStruct((B, _DOUT), jnp.float32),
        scratch_shapes=[pltpu.VMEM((_TB, _DF), jnp.float32)],
        compiler_params=pltpu.CompilerParams(
            dimension_semantics=("arbitrary", "arbitrary"),
        ),
    )(
        state.astype(jnp.bfloat16),
        c3,
        wg,
        W1.astype(jnp.bfloat16),
        b1.reshape(_E, 1, _DF),
        W2.astype(jnp.bfloat16),
        b2.reshape(_E, 1, _DF),
        hw_cat.astype(jnp.bfloat16),
        hb_cat,
    )
    return out


# speedup vs baseline: 1.1994x; 1.1994x over previous
import functools

import jax
import jax.numpy as jnp
from jax.experimental import pallas as pl
from jax.experimental.pallas import tpu as pltpu

_E = 8
_C = 10
_DIN = 768
_DF = 768
_DOUT = 64
_TB = 2048


def _fused_body(state_ref, c_ref, wg_ref, w1_ref, b1_ref, w2_ref, b2_ref,
                hw_ref, hb_ref, out_ref, acc_ref):
    e = pl.program_id(1)

    x = state_ref[...]
    h = jnp.dot(x, w1_ref[0], preferred_element_type=jnp.float32)
    h = jnp.maximum(h + b1_ref[0], 0.0).astype(jnp.bfloat16)
    f = jnp.dot(h, w2_ref[0], preferred_element_type=jnp.float32)
    f = jnp.maximum(f + b2_ref[0], 0.0)

    c_col = c_ref[0]
    gate = wg_ref[0, 0]

    contrib = gate * f

    @pl.when(e == 0)
    def _init():
        acc_ref[...] = contrib

    @pl.when(e > 0)
    def _accum():
        acc_ref[...] += contrib

    @pl.when(e == _E - 1)
    def _heads():
        mixed = jnp.maximum(acc_ref[...], 0.0).astype(jnp.bfloat16)
        all_heads = jnp.dot(mixed, hw_ref[...],
                            preferred_element_type=jnp.float32)
        all_heads = all_heads + hb_ref[...]
        out = jnp.zeros((_TB, _DOUT), jnp.float32)
        for ci in range(_C):
            sl = all_heads[:, ci * _DOUT:(ci + 1) * _DOUT]
            out = out + jnp.where(c_col == ci, sl, 0.0)
        out_ref[...] = out


@functools.partial(jax.jit, static_argnames=())
def kernel(state, c, W_task, W1, b1, W2, b2, head_W, head_b):
    B = state.shape[0]
    nb = B // _TB
    c3 = c.astype(jnp.int32).reshape(nb, _TB, 1)
    oh = jax.nn.one_hot(c, _C, dtype=jnp.float32)
    wg = oh @ W_task.T
    wg = wg.T.reshape(_E, nb, _TB, 1).transpose(1, 0, 2, 3)
    hw_cat = jnp.transpose(head_W, (1, 0, 2)).reshape(_DF, _C * _DOUT)
    hb_cat = head_b.reshape(1, _C * _DOUT)

    out = pl.pallas_call(
        _fused_body,
        grid=(nb, _E),
        in_specs=[
            pl.BlockSpec((_TB, _DIN), lambda ib, e: (ib, 0)),
            pl.BlockSpec((1, _TB, 1), lambda ib, e: (ib, 0, 0)),
            pl.BlockSpec((1, 1, _TB, 1), lambda ib, e: (ib, e, 0, 0)),
            pl.BlockSpec((1, _DIN, _DF), lambda ib, e: (e, 0, 0)),
            pl.BlockSpec((1, 1, _DF), lambda ib, e: (e, 0, 0)),
            pl.BlockSpec((1, _DF, _DF), lambda ib, e: (e, 0, 0)),
            pl.BlockSpec((1, 1, _DF), lambda ib, e: (e, 0, 0)),
            pl.BlockSpec((_DF, _C * _DOUT), lambda ib, e: (0, 0)),
            pl.BlockSpec((1, _C * _DOUT), lambda ib, e: (0, 0)),
        ],
        out_specs=pl.BlockSpec((_TB, _DOUT), lambda ib, e: (ib, 0)),
        out_shape=jax.ShapeDtypeStruct((B, _DOUT), jnp.float32),
        scratch_shapes=[pltpu.VMEM((_TB, _DF), jnp.float32)],
        compiler_params=pltpu.CompilerParams(
            dimension_semantics=("arbitrary", "arbitrary"),
        ),
    )(
        state.astype(jnp.bfloat16),
        c3,
        wg,
        W1.astype(jnp.bfloat16),
        b1.reshape(_E, 1, _DF),
        W2.astype(jnp.bfloat16),
        b2.reshape(_E, 1, _DF),
        hw_cat.astype(jnp.bfloat16),
        hb_cat,
    )
    return out


# speedup vs baseline: 1.5155x; 1.2635x over previous
import functools

import jax
import jax.numpy as jnp
from jax.experimental import pallas as pl
from jax.experimental.pallas import tpu as pltpu

_E = 8
_C = 10
_DIN = 768
_DF = 768
_DOUT = 64
_TB = 2048


def _fused_body(state_ref, c_ref, wtask_ref, w1_ref, b1_ref, w2_ref, b2_ref,
                hw_ref, hb_ref, out_ref, acc_ref):
    e = pl.program_id(1)

    x = state_ref[...]
    h = jnp.dot(x, w1_ref[0], preferred_element_type=jnp.float32)
    h = jnp.maximum(h + b1_ref[0], 0.0)
    f = jnp.dot(h, w2_ref[0], preferred_element_type=jnp.float32)
    f = jnp.maximum(f + b2_ref[0], 0.0)

    c_col = c_ref[0]
    wt = wtask_ref[...]
    sel_e = jax.lax.broadcasted_iota(jnp.int32, (_E, _C), 0) == e
    row = jnp.sum(jnp.where(sel_e, wt, 0.0), axis=0, keepdims=True)
    oh = c_col == jax.lax.broadcasted_iota(jnp.int32, (_TB, _C), 1)
    gate = jnp.sum(jnp.where(oh, row, 0.0), axis=1, keepdims=True)

    contrib = gate * f

    @pl.when(e == 0)
    def _init():
        acc_ref[...] = contrib

    @pl.when(e > 0)
    def _accum():
        acc_ref[...] += contrib

    @pl.when(e == _E - 1)
    def _heads():
        mixed = jnp.maximum(acc_ref[...], 0.0)
        all_heads = jnp.dot(mixed, hw_ref[...],
                            preferred_element_type=jnp.float32)
        all_heads = all_heads + hb_ref[...]
        out = jnp.zeros((_TB, _DOUT), jnp.float32)
        for ci in range(_C):
            sl = all_heads[:, ci * _DOUT:(ci + 1) * _DOUT]
            out = out + jnp.where(c_col == ci, sl, 0.0)
        out_ref[...] = out


@functools.partial(jax.jit, static_argnames=())
def kernel(state, c, W_task, W1, b1, W2, b2, head_W, head_b):
    B = state.shape[0]
    nb = B // _TB
    c3 = c.astype(jnp.int32).reshape(nb, _TB, 1)
    hw_cat = jnp.transpose(head_W, (1, 0, 2)).reshape(_DF, _C * _DOUT)
    hb_cat = head_b.reshape(1, _C * _DOUT)

    out = pl.pallas_call(
        _fused_body,
        grid=(nb, _E),
        in_specs=[
            pl.BlockSpec((_TB, _DIN), lambda ib, e: (ib, 0)),
            pl.BlockSpec((1, _TB, 1), lambda ib, e: (ib, 0, 0)),
            pl.BlockSpec((_E, _C), lambda ib, e: (0, 0)),
            pl.BlockSpec((1, _DIN, _DF), lambda ib, e: (e, 0, 0)),
            pl.BlockSpec((1, 1, _DF), lambda ib, e: (e, 0, 0)),
            pl.BlockSpec((1, _DF, _DF), lambda ib, e: (e, 0, 0)),
            pl.BlockSpec((1, 1, _DF), lambda ib, e: (e, 0, 0)),
            pl.BlockSpec((_DF, _C * _DOUT), lambda ib, e: (0, 0)),
            pl.BlockSpec((1, _C * _DOUT), lambda ib, e: (0, 0)),
        ],
        out_specs=pl.BlockSpec((_TB, _DOUT), lambda ib, e: (ib, 0)),
        out_shape=jax.ShapeDtypeStruct((B, _DOUT), jnp.float32),
        scratch_shapes=[pltpu.VMEM((_TB, _DF), jnp.float32)],
        compiler_params=pltpu.CompilerParams(
            dimension_semantics=("arbitrary", "arbitrary"),
        ),
    )(
        state,
        c3,
        W_task,
        W1,
        b1.reshape(_E, 1, _DF),
        W2,
        b2.reshape(_E, 1, _DF),
        hw_cat,
        hb_cat,
    )
    return out


# speedup vs baseline: 1.5746x; 1.0390x over previous
import functools

import jax
import jax.numpy as jnp
from jax.experimental import pallas as pl
from jax.experimental.pallas import tpu as pltpu

_E = 8
_EP = 2
_C = 10
_DIN = 768
_DF = 768
_DOUT = 64
_TB = 1024


def _fused_body(state_ref, c_ref, wtask_ref, w1_ref, b1_ref, w2_ref, b2_ref,
                hw_ref, hb_ref, out_ref, acc_ref):
    je = pl.program_id(1)
    c_col = c_ref[0]
    wt = wtask_ref[...]
    oh = c_col == jax.lax.broadcasted_iota(jnp.int32, (_TB, _C), 1)
    x = state_ref[...]

    contrib = jnp.zeros((_TB, _DF), jnp.float32)
    for k in range(_EP):
        h = jnp.dot(x, w1_ref[k], preferred_element_type=jnp.float32)
        h = jnp.maximum(h + b1_ref[k], 0.0)
        f = jnp.dot(h, w2_ref[k], preferred_element_type=jnp.float32)
        f = jnp.maximum(f + b2_ref[k], 0.0)
        e = je * _EP + k
        sel_e = jax.lax.broadcasted_iota(jnp.int32, (_E, _C), 0) == e
        row = jnp.sum(jnp.where(sel_e, wt, 0.0), axis=0, keepdims=True)
        gate = jnp.sum(jnp.where(oh, row, 0.0), axis=1, keepdims=True)
        contrib = contrib + gate * f

    @pl.when(je == 0)
    def _init():
        acc_ref[...] = contrib

    @pl.when(je > 0)
    def _accum():
        acc_ref[...] += contrib

    @pl.when(je == _E // _EP - 1)
    def _heads():
        mixed = jnp.maximum(acc_ref[...], 0.0)
        all_heads = jnp.dot(mixed, hw_ref[...],
                            preferred_element_type=jnp.float32)
        all_heads = all_heads + hb_ref[...]
        out = jnp.zeros((_TB, _DOUT), jnp.float32)
        for ci in range(_C):
            sl = all_heads[:, ci * _DOUT:(ci + 1) * _DOUT]
            out = out + jnp.where(c_col == ci, sl, 0.0)
        out_ref[...] = out


@functools.partial(jax.jit, static_argnames=())
def kernel(state, c, W_task, W1, b1, W2, b2, head_W, head_b):
    B = state.shape[0]
    nb = B // _TB
    c3 = c.astype(jnp.int32).reshape(nb, _TB, 1)
    hw_cat = jnp.transpose(head_W, (1, 0, 2)).reshape(_DF, _C * _DOUT)
    hb_cat = head_b.reshape(1, _C * _DOUT)

    out = pl.pallas_call(
        _fused_body,
        grid=(nb, _E // _EP),
        in_specs=[
            pl.BlockSpec((_TB, _DIN), lambda ib, je: (ib, 0)),
            pl.BlockSpec((1, _TB, 1), lambda ib, je: (ib, 0, 0)),
            pl.BlockSpec((_E, _C), lambda ib, je: (0, 0)),
            pl.BlockSpec((_EP, _DIN, _DF), lambda ib, je: (je, 0, 0)),
            pl.BlockSpec((_EP, 1, _DF), lambda ib, je: (je, 0, 0)),
            pl.BlockSpec((_EP, _DF, _DF), lambda ib, je: (je, 0, 0)),
            pl.BlockSpec((_EP, 1, _DF), lambda ib, je: (je, 0, 0)),
            pl.BlockSpec((_DF, _C * _DOUT), lambda ib, je: (0, 0)),
            pl.BlockSpec((1, _C * _DOUT), lambda ib, je: (0, 0)),
        ],
        out_specs=pl.BlockSpec((_TB, _DOUT), lambda ib, je: (ib, 0)),
        out_shape=jax.ShapeDtypeStruct((B, _DOUT), jnp.float32),
        scratch_shapes=[pltpu.VMEM((_TB, _DF), jnp.float32)],
        compiler_params=pltpu.CompilerParams(
            dimension_semantics=("arbitrary", "arbitrary"),
        ),
    )(
        state,
        c3,
        W_task,
        W1,
        b1.reshape(_E, 1, _DF),
        W2,
        b2.reshape(_E, 1, _DF),
        hw_cat,
        hb_cat,
    )
    return out


# speedup vs baseline: 1.6656x; 1.0578x over previous
import functools

import jax
import jax.numpy as jnp
from jax.experimental import pallas as pl
from jax.experimental.pallas import tpu as pltpu

_E = 8
_C = 10
_DIN = 768
_DF = 768
_DOUT = 64
_TB = 512


def _fused_body(state_ref, c_ref, wtask_ref, w1_ref, b1_ref, w2_ref, b2_ref,
                hw_ref, hb_ref, out_ref):
    c_col = c_ref[0]
    wt = wtask_ref[...]
    oh = c_col == jax.lax.broadcasted_iota(jnp.int32, (_TB, _C), 1)
    x = state_ref[...]

    acc = jnp.zeros((_TB, _DF), jnp.float32)
    for e in range(_E):
        h = jnp.dot(x, w1_ref[e], preferred_element_type=jnp.float32)
        h = jnp.maximum(h + b1_ref[e], 0.0)
        f = jnp.dot(h, w2_ref[e], preferred_element_type=jnp.float32)
        f = jnp.maximum(f + b2_ref[e], 0.0)
        gate = jnp.sum(jnp.where(oh, wt[e:e + 1, :], 0.0),
                       axis=1, keepdims=True)
        acc = acc + gate * f

    mixed = jnp.maximum(acc, 0.0)
    all_heads = jnp.dot(mixed, hw_ref[...],
                        preferred_element_type=jnp.float32)
    all_heads = all_heads + hb_ref[...]
    out = jnp.zeros((_TB, _DOUT), jnp.float32)
    for ci in range(_C):
        sl = all_heads[:, ci * _DOUT:(ci + 1) * _DOUT]
        out = out + jnp.where(c_col == ci, sl, 0.0)
    out_ref[...] = out


@functools.partial(jax.jit, static_argnames=())
def kernel(state, c, W_task, W1, b1, W2, b2, head_W, head_b):
    B = state.shape[0]
    nb = B // _TB
    c3 = c.astype(jnp.int32).reshape(nb, _TB, 1)
    hw_cat = jnp.transpose(head_W, (1, 0, 2)).reshape(_DF, _C * _DOUT)
    hb_cat = head_b.reshape(1, _C * _DOUT)

    out = pl.pallas_call(
        _fused_body,
        grid=(nb,),
        in_specs=[
            pl.BlockSpec((_TB, _DIN), lambda ib: (ib, 0)),
            pl.BlockSpec((1, _TB, 1), lambda ib: (ib, 0, 0)),
            pl.BlockSpec((_E, _C), lambda ib: (0, 0)),
            pl.BlockSpec((_E, _DIN, _DF), lambda ib: (0, 0, 0)),
            pl.BlockSpec((_E, 1, _DF), lambda ib: (0, 0, 0)),
            pl.BlockSpec((_E, _DF, _DF), lambda ib: (0, 0, 0)),
            pl.BlockSpec((_E, 1, _DF), lambda ib: (0, 0, 0)),
            pl.BlockSpec((_DF, _C * _DOUT), lambda ib: (0, 0)),
            pl.BlockSpec((1, _C * _DOUT), lambda ib: (0, 0)),
        ],
        out_specs=pl.BlockSpec((_TB, _DOUT), lambda ib: (ib, 0)),
        out_shape=jax.ShapeDtypeStruct((B, _DOUT), jnp.float32),
        compiler_params=pltpu.CompilerParams(
            dimension_semantics=("arbitrary",),
        ),
    )(
        state,
        c3,
        W_task,
        W1,
        b1.reshape(_E, 1, _DF),
        W2,
        b2.reshape(_E, 1, _DF),
        hw_cat,
        hb_cat,
    )
    return out


# speedup vs baseline: 1.6837x; 1.0108x over previous
import functools

import jax
import jax.numpy as jnp
from jax.experimental import pallas as pl
from jax.experimental.pallas import tpu as pltpu

_E = 8
_C = 10
_DIN = 768
_DF = 768
_DOUT = 64
_TB = 1024


def _fused_body(state_ref, c_ref, wtask_ref, w1_ref, b1_ref, w2_ref, b2_ref,
                hw_ref, hb_ref, out_ref):
    c_col = c_ref[0]
    wt = wtask_ref[...]
    oh = c_col == jax.lax.broadcasted_iota(jnp.int32, (_TB, _C), 1)
    x = state_ref[...]

    acc = jnp.zeros((_TB, _DF), jnp.float32)
    for e in range(_E):
        h = jnp.dot(x, w1_ref[e], preferred_element_type=jnp.float32)
        h = jnp.maximum(h + b1_ref[e], 0.0)
        f = jnp.dot(h, w2_ref[e], preferred_element_type=jnp.float32)
        f = jnp.maximum(f + b2_ref[e], 0.0)
        gate = jnp.sum(jnp.where(oh, wt[e:e + 1, :], 0.0),
                       axis=1, keepdims=True)
        acc = acc + gate * f

    mixed = jnp.maximum(acc, 0.0)
    all_heads = jnp.dot(mixed, hw_ref[...],
                        preferred_element_type=jnp.float32)
    all_heads = all_heads + hb_ref[...]
    out = jnp.zeros((_TB, _DOUT), jnp.float32)
    for ci in range(_C):
        sl = all_heads[:, ci * _DOUT:(ci + 1) * _DOUT]
        out = out + jnp.where(c_col == ci, sl, 0.0)
    out_ref[...] = out


@functools.partial(jax.jit, static_argnames=())
def kernel(state, c, W_task, W1, b1, W2, b2, head_W, head_b):
    B = state.shape[0]
    nb = B // _TB
    c3 = c.astype(jnp.int32).reshape(nb, _TB, 1)
    hw_cat = jnp.transpose(head_W, (1, 0, 2)).reshape(_DF, _C * _DOUT)
    hb_cat = head_b.reshape(1, _C * _DOUT)

    out = pl.pallas_call(
        _fused_body,
        grid=(nb,),
        in_specs=[
            pl.BlockSpec((_TB, _DIN), lambda ib: (ib, 0)),
            pl.BlockSpec((1, _TB, 1), lambda ib: (ib, 0, 0)),
            pl.BlockSpec((_E, _C), lambda ib: (0, 0)),
            pl.BlockSpec((_E, _DIN, _DF), lambda ib: (0, 0, 0)),
            pl.BlockSpec((_E, 1, _DF), lambda ib: (0, 0, 0)),
            pl.BlockSpec((_E, _DF, _DF), lambda ib: (0, 0, 0)),
            pl.BlockSpec((_E, 1, _DF), lambda ib: (0, 0, 0)),
            pl.BlockSpec((_DF, _C * _DOUT), lambda ib: (0, 0)),
            pl.BlockSpec((1, _C * _DOUT), lambda ib: (0, 0)),
        ],
        out_specs=pl.BlockSpec((_TB, _DOUT), lambda ib: (ib, 0)),
        out_shape=jax.ShapeDtypeStruct((B, _DOUT), jnp.float32),
        compiler_params=pltpu.CompilerParams(
            dimension_semantics=("arbitrary",),
        ),
    )(
        state,
        c3,
        W_task,
        W1,
        b1.reshape(_E, 1, _DF),
        W2,
        b2.reshape(_E, 1, _DF),
        hw_cat,
        hb_cat,
    )
    return out
